# baseline (speedup 1.0000x reference)
import functools

import jax
import jax.numpy as jnp
from jax import lax
from jax.experimental import pallas as pl
from jax.experimental.pallas import tpu as pltpu
from jax.experimental.pallas import tpu_sc as plsc

N = 10000
E = 320000
CH = 128
OUT_CH = 64
NUM_GRAPHS = 64

NC = 2
NS = 16
NW = NC * NS
K = 96

E_TOT = E + N
G = 8
S = G * K
NCH1 = -(-(-(-E_TOT // (NW * K))) // (2 * G)) * 2 * G
NCH2 = NCH1
NSUP2 = NCH2 // G
NSUP = NCH1 // G
E_PAD = NW * K * NCH1
EPW1 = K * NCH1
EPW2 = K * NCH2

N_PAD = 10240
ROWS_PT = N_PAD // NS
DEN_PT = N_PAD // NS

def _sc_layer1_body(src_hbm, dst_hbm, x_hbm, tst_hbm, zflat_hbm, out_hbm,
                    x_v, ts_v, td_v,
                    sbigA, dbigA, wgA, idxgA, semLA, semSA,
                    sbigB, dbigB, wgB, idxgB, semLB, semSB,
                    c_acc):
    cid = lax.axis_index("c")
    sid = lax.axis_index("s")
    wid = cid * NS + sid

    pltpu.sync_copy(x_hbm, x_v)
    pltpu.sync_copy(tst_hbm.at[0], ts_v)
    pltpu.sync_copy(tst_hbm.at[1], td_v)
    seg = (N * CH) // NS
    pltpu.sync_copy(zflat_hbm.at[pl.ds(sid * seg, seg)],
                    c_acc.at[pl.ds(sid * seg, seg)])
    plsc.subcore_barrier()

    def issue_loads(u, sb, db, semL):
        base = wid * EPW1 + u * S
        pltpu.async_copy(src_hbm.at[pl.ds(base, S)], sb, semL)
        pltpu.async_copy(dst_hbm.at[pl.ds(base, S)], db, semL)

    def wait_loads(u, sb, db, semL):
        base = wid * EPW1 + u * S
        pltpu.make_async_copy(src_hbm.at[pl.ds(base, S)], sb, semL).wait()
        pltpu.make_async_copy(dst_hbm.at[pl.ds(base, S)], db, semL).wait()

    def process_super(u, sb, db, wg, idxg, semS):
        base = wid * EPW1 + u * S
        for g in range(G):
            for j in range(K // 16):
                off = g * K + j * 16
                sl = sb[pl.ds(off, 16)]
                d = db[pl.ds(off, 16)]
                cs = plsc.load_gather(x_v, [sl])
                cd = plsc.load_gather(x_v, [d])
                a = plsc.load_gather(ts_v, [cs])
                b = plsc.load_gather(td_v, [cd])
                e = a + b
                e = jnp.where(e > 0.0, e, 0.2 * e)
                w = jnp.exp(e)
                eid = base + off + lax.iota(jnp.int32, 16)
                w = jnp.where(eid < E_TOT, w, 0.0)
                wg[g, pl.ds(j * 16, 16)] = w
                idxg[g, pl.ds(j * 16, 16)] = d * CH + cs
            pltpu.async_copy(wg.at[g], c_acc.at[idxg.at[g]], semS, add=True)

    def drain_scatters(wg, idxg, semS):
        for g in range(G):
            pltpu.make_async_copy(wg.at[g], c_acc.at[idxg.at[g]], semS).wait()

    issue_loads(0, sbigA, dbigA, semLA)

    def pair(u2, carry):
        uA = u2 * 2
        uB = uA + 1
        uA2 = lax.rem(uA + 2, NSUP)
        issue_loads(uB, sbigB, dbigB, semLB)
        wait_loads(uA, sbigA, dbigA, semLA)
        process_super(uA, sbigA, dbigA, wgA, idxgA, semSA)
        issue_loads(uA2, sbigA, dbigA, semLA)
        drain_scatters(wgA, idxgA, semSA)
        wait_loads(uB, sbigB, dbigB, semLB)
        process_super(uB, sbigB, dbigB, wgB, idxgB, semSB)
        drain_scatters(wgB, idxgB, semSB)
        return carry

    lax.fori_loop(0, NSUP // 2, pair, 0)
    wait_loads(0, sbigA, dbigA, semLA)
    plsc.subcore_barrier()
    pltpu.sync_copy(c_acc.at[pl.ds(sid * seg, seg)],
                    out_hbm.at[cid, pl.ds(sid * seg, seg)])


def _sc_layer2_body(src2d_hbm, dst2d_hbm, as_hbm, ad_hbm, hw2_hbm, zrows_hbm,
                    zden_hbm, num_out, den_out,
                    as_v, ad_v,
                    sbigA, dbigA, semLA, sbigB, dbigB, semLB,
                    rows0, semG0, rows1, semG1,
                    wg, semS0, semS1,
                    num_acc, den_acc):
    cid = lax.axis_index("c")
    sid = lax.axis_index("s")
    wid = cid * NS + sid
    rowbase = wid * NCH2

    pltpu.sync_copy(as_hbm, as_v)
    pltpu.sync_copy(ad_hbm, ad_v)

    @pl.when(sid < NS - 1)
    def _():
        pltpu.sync_copy(zrows_hbm.at[pl.ds(sid * 624, 624)],
                        num_acc.at[pl.ds(sid * 624, 624)])

    @pl.when(sid == NS - 1)
    def _():
        pltpu.sync_copy(zrows_hbm.at[pl.ds((NS - 1) * 624, 640)],
                        num_acc.at[pl.ds((NS - 1) * 624, 640)])

    pltpu.sync_copy(zden_hbm.at[pl.ds(sid * DEN_PT, DEN_PT)],
                    den_acc.at[pl.ds(sid * DEN_PT, DEN_PT)])
    plsc.subcore_barrier()

    rows = (rows0, rows1)
    semG = (semG0, semG1)
    semS = (semS0, semS1)

    def issue_loads(u, sb, db, semL):
        pltpu.async_copy(src2d_hbm.at[pl.ds(rowbase + u * G, G)], sb, semL)
        pltpu.async_copy(dst2d_hbm.at[pl.ds(rowbase + u * G, G)], db, semL)

    def wait_loads(u, sb, db, semL):
        pltpu.make_async_copy(src2d_hbm.at[pl.ds(rowbase + u * G, G)],
                              sb, semL).wait()
        pltpu.make_async_copy(dst2d_hbm.at[pl.ds(rowbase + u * G, G)],
                              db, semL).wait()

    def issue_gather(sb, g, p):
        pltpu.async_copy(hw2_hbm.at[sb.at[g]], rows[p], semG[p])

    def wait_gather(sb, g, p):
        pltpu.make_async_copy(hw2_hbm.at[sb.at[g]], rows[p], semG[p]).wait()

    def issue_scatter(db, g, p):
        pltpu.async_copy(rows[p], num_acc.at[db.at[g]], semS[p], add=True)
        pltpu.async_copy(wg.at[g], den_acc.at[db.at[g]], semS[p], add=True)

    def drain_scatter(db, g, p):
        pltpu.make_async_copy(rows[p], num_acc.at[db.at[g]], semS[p]).wait()
        pltpu.make_async_copy(wg.at[g], den_acc.at[db.at[g]], semS[p]).wait()

    def process_super(u, sb, db, next_info, skip0_pred, issue_next_fn):
        u_next, sb_next, db_next, semL_next = next_info
        base = wid * EPW2 + u * G * K
        for g in range(G):
            p = g % 2
            for j in range(K // 16):
                sl = sb[g, pl.ds(j * 16, 16)]
                d = db[g, pl.ds(j * 16, 16)]
                a = plsc.load_gather(as_v, [sl])
                b = plsc.load_gather(ad_v, [d])
                e = a + b
                e = jnp.where(e > 0.0, e, 0.2 * e)
                w = jnp.exp(e)
                eid = base + g * K + j * 16 + lax.iota(jnp.int32, 16)
                w = jnp.where(eid < E_TOT, w, 0.0)
                wg[g, pl.ds(j * 16, 16)] = w
            if g == 0:
                if skip0_pred is not None:
                    @pl.when(skip0_pred)
                    def _():
                        drain_scatter(db, g, 1 - p)
                else:
                    drain_scatter(db, g, 1 - p)
                issue_next_fn()
            else:
                drain_scatter(db, g, 1 - p)
            if g < G - 1:
                issue_gather(sb, g + 1, 1 - p)
            else:
                wait_loads(u_next, sb_next, db_next, semL_next)
                issue_gather(sb_next, 0, 1 - p)
            wait_gather(sb, g, p)

            def scale_row(r, carry2):
                gi = jnp.full((16,), g, jnp.int32)
                wr = plsc.load_gather(wg, [gi, jnp.full((16,), r, jnp.int32)])
                for l in range(CH // 16):
                    rows[p][r, pl.ds(l * 16, 16)] = (
                        rows[p][r, pl.ds(l * 16, 16)] * wr)
                return carry2

            lax.fori_loop(0, K, scale_row, 0, unroll=8)
            issue_scatter(db, g, p)

    issue_loads(0, sbigA, dbigA, semLA)
    wait_loads(0, sbigA, dbigA, semLA)
    issue_gather(sbigA, 0, 0)

    def pairloop(i, carry):
        uA = i * 2
        uB = uA + 1
        uA2 = lax.rem(uA + 2, NSUP2)
        process_super(uA, sbigA, dbigA, (uB, sbigB, dbigB, semLB),
                      skip0_pred=(i > 0),
                      issue_next_fn=lambda: issue_loads(uB, sbigB, dbigB,
                                                        semLB))
        process_super(uB, sbigB, dbigB, (uA2, sbigA, dbigA, semLA),
                      skip0_pred=None,
                      issue_next_fn=lambda: issue_loads(uA2, sbigA, dbigA,
                                                        semLA))
        return carry

    lax.fori_loop(0, NSUP2 // 2, pairloop, 0)
    drain_scatter(dbigA, 0, 1)
    pltpu.make_async_copy(hw2_hbm.at[sbigA.at[0]], rows0, semG0).wait()
    plsc.subcore_barrier()

    @pl.when(sid < NS - 1)
    def _():
        pltpu.sync_copy(num_acc.at[pl.ds(sid * 624, 624)],
                        num_out.at[cid, pl.ds(sid * 624, 624)])

    @pl.when(sid == NS - 1)
    def _():
        pltpu.sync_copy(num_acc.at[pl.ds((NS - 1) * 624, 640)],
                        num_out.at[cid, pl.ds((NS - 1) * 624, 640)])

    pltpu.sync_copy(den_acc.at[pl.ds(sid * DEN_PT, DEN_PT)],
                    den_out.at[cid, pl.ds(sid * DEN_PT, DEN_PT)])


@functools.lru_cache(maxsize=None)
def _sc_kernels():
    mesh = plsc.VectorSubcoreMesh(core_axis_name="c", subcore_axis_name="s",
                                  num_cores=NC, num_subcores=NS)
    cparams = pltpu.CompilerParams(needs_layout_passes=False)
    layer1 = pl.kernel(
        _sc_layer1_body,
        out_type=jax.ShapeDtypeStruct((NC, N * CH), jnp.float32),
        mesh=mesh,
        compiler_params=cparams,
        scratch_types=[
            pltpu.VMEM((N,), jnp.int32),
            pltpu.VMEM((CH,), jnp.float32),
            pltpu.VMEM((CH,), jnp.float32),
            pltpu.VMEM((S,), jnp.int32),
            pltpu.VMEM((S,), jnp.int32),
            pltpu.VMEM((G, K), jnp.float32),
            pltpu.VMEM((G, K), jnp.int32),
            pltpu.SemaphoreType.DMA,
            pltpu.SemaphoreType.DMA,
            pltpu.VMEM((S,), jnp.int32),
            pltpu.VMEM((S,), jnp.int32),
            pltpu.VMEM((G, K), jnp.float32),
            pltpu.VMEM((G, K), jnp.int32),
            pltpu.SemaphoreType.DMA,
            pltpu.SemaphoreType.DMA,
            pltpu.VMEM_SHARED((N * CH,), jnp.float32),
        ],
    )
    layer2 = pl.kernel(
        _sc_layer2_body,
        out_type=[jax.ShapeDtypeStruct((NC, N, CH), jnp.float32),
                  jax.ShapeDtypeStruct((NC, N_PAD), jnp.float32)],
        mesh=mesh,
        compiler_params=cparams,
        scratch_types=[
            pltpu.VMEM((N,), jnp.float32),
            pltpu.VMEM((N,), jnp.float32),
            pltpu.VMEM((G, K), jnp.int32),
            pltpu.VMEM((G, K), jnp.int32),
            pltpu.SemaphoreType.DMA,
            pltpu.VMEM((G, K), jnp.int32),
            pltpu.VMEM((G, K), jnp.int32),
            pltpu.SemaphoreType.DMA,
            pltpu.VMEM((K, CH), jnp.float32),
            pltpu.SemaphoreType.DMA,
            pltpu.VMEM((K, CH), jnp.float32),
            pltpu.SemaphoreType.DMA,
            pltpu.VMEM((G, K), jnp.float32),
            pltpu.SemaphoreType.DMA,
            pltpu.SemaphoreType.DMA,
            pltpu.VMEM_SHARED((N, CH), jnp.float32),
            pltpu.VMEM_SHARED((N_PAD,), jnp.float32),
        ],
    )
    return layer1, layer2


def _prep_body(emb_ref, w1_ref, as1_ref, ad1_ref, tst_ref, m1_ref):
    m1 = jnp.dot(emb_ref[...], w1_ref[...], preferred_element_type=jnp.float32)
    m1_ref[...] = m1
    dn = (((1,), (1,)), ((), ()))
    tst_ref[0:1, :] = lax.dot_general(as1_ref[...], m1, dn,
                                      preferred_element_type=jnp.float32)
    tst_ref[1:2, :] = lax.dot_general(ad1_ref[...], m1, dn,
                                      preferred_element_type=jnp.float32)


def _tc_prep(emb, w1, as1, ad1):
    return pl.pallas_call(
        _prep_body,
        out_shape=[jax.ShapeDtypeStruct((2, CH), jnp.float32),
                   jax.ShapeDtypeStruct((CH, CH), jnp.float32)],
    )(emb, w1, as1, ad1)


R = 1000
NB = N // R


def _mid_body(c_ref, m1_ref, b1_ref, w2_ref, as2_ref, ad2_ref,
              hw2_ref, aux_ref):
    c = c_ref[0] + c_ref[1]
    den = jnp.sum(c, axis=1, keepdims=True) + 1e-16
    h1 = jnp.dot(c, m1_ref[...], preferred_element_type=jnp.float32) / den
    h1 = h1 + b1_ref[...]
    h2 = jnp.maximum(h1, 0.0)
    hw2 = jnp.dot(h2, w2_ref[...], preferred_element_type=jnp.float32)
    hw2_ref[...] = hw2
    a_s = jnp.sum(hw2 * as2_ref[...], axis=1, keepdims=True)
    a_d = jnp.sum(hw2 * ad2_ref[...], axis=1, keepdims=True)
    lane = lax.broadcasted_iota(jnp.int32, (1, CH), 1)
    aux_ref[...] = (jnp.where(lane == 0, a_s, 0.0)
                    + jnp.where(lane == 1, a_d, 0.0))


def _tc_mid(c3, m1, b1, w2, as2, ad2):
    return pl.pallas_call(
        _mid_body,
        grid=(NB,),
        in_specs=[
            pl.BlockSpec((NC, R, CH), lambda i: (0, i, 0)),
            pl.BlockSpec((CH, CH), lambda i: (0, 0)),
            pl.BlockSpec((1, CH), lambda i: (0, 0)),
            pl.BlockSpec((CH, CH), lambda i: (0, 0)),
            pl.BlockSpec((1, CH), lambda i: (0, 0)),
            pl.BlockSpec((1, CH), lambda i: (0, 0)),
        ],
        out_specs=[
            pl.BlockSpec((R, CH), lambda i: (i, 0)),
            pl.BlockSpec((R, CH), lambda i: (i, 0)),
        ],
        out_shape=[jax.ShapeDtypeStruct((N, CH), jnp.float32),
                   jax.ShapeDtypeStruct((N, CH), jnp.float32)],
    )(c3, m1, b1, w2, as2, ad2)


def _fin_body(num_ref, den_ref, b2_ref, batch_ref, wo_ref, bo_ref, out_ref,
              acc_sum, acc_cnt):
    i = pl.program_id(0)

    @pl.when(i == 0)
    def _():
        acc_sum[...] = jnp.zeros_like(acc_sum)
        acc_cnt[...] = jnp.zeros_like(acc_cnt)

    h = (num_ref[0] + num_ref[1]) / den_ref[...] + b2_ref[...]
    bvec = batch_ref[0]
    gids = lax.broadcasted_iota(jnp.int32, (NUM_GRAPHS, R), 0)
    p = (gids == bvec).astype(jnp.float32)
    acc_sum[...] += jnp.dot(p, h, preferred_element_type=jnp.float32)
    acc_cnt[...] += jnp.sum(p, axis=1, keepdims=True)

    @pl.when(i == NB - 1)
    def _():
        pooled = acc_sum[...] / jnp.maximum(acc_cnt[...], 1.0)
        out_ref[...] = (jnp.dot(pooled, wo_ref[...],
                                preferred_element_type=jnp.float32)
                        + bo_ref[...])


def _tc_final(num3, den_b, b2, batch3, wo_pad, bo_pad):
    return pl.pallas_call(
        _fin_body,
        grid=(NB,),
        in_specs=[
            pl.BlockSpec((NC, R, CH), lambda i: (0, i, 0)),
            pl.BlockSpec((R, CH), lambda i: (i, 0)),
            pl.BlockSpec((1, CH), lambda i: (0, 0)),
            pl.BlockSpec((1, 1, R), lambda i: (i, 0, 0)),
            pl.BlockSpec((CH, CH), lambda i: (0, 0)),
            pl.BlockSpec((1, CH), lambda i: (0, 0)),
        ],
        out_specs=pl.BlockSpec((NUM_GRAPHS, CH), lambda i: (0, 0)),
        out_shape=jax.ShapeDtypeStruct((NUM_GRAPHS, CH), jnp.float32),
        scratch_shapes=[pltpu.VMEM((NUM_GRAPHS, CH), jnp.float32),
                        pltpu.VMEM((NUM_GRAPHS, CH), jnp.float32)],
    )(num3, den_b, b2, batch3, wo_pad, bo_pad)


def kernel(x, edge_index, batch, emb_table, W1, a_src1, a_dst1, b1,
           W2, a_src2, a_dst2, b2, W_out, b_out):
    x = x.astype(jnp.int32)
    loop = jnp.arange(N, dtype=jnp.int32)
    padz = jnp.zeros((E_PAD - E_TOT,), jnp.int32)
    src = jnp.concatenate([edge_index[0].astype(jnp.int32), loop, padz])
    dst = jnp.concatenate([edge_index[1].astype(jnp.int32), loop, padz])

    sc_layer1, sc_layer2 = _sc_kernels()

    tst, m1 = _tc_prep(emb_table, W1,
                       a_src1.reshape(1, CH), a_dst1.reshape(1, CH))

    zflat = jnp.zeros((N * CH,), jnp.float32)
    c_part = sc_layer1(src, dst, x, tst, zflat)

    hw2, aux = _tc_mid(c_part.reshape(NC, N, CH), m1, b1.reshape(1, CH),
                       W2, a_src2.reshape(1, CH), a_dst2.reshape(1, CH))

    zrows = jnp.zeros((N, CH), jnp.float32)
    zden = jnp.zeros((N_PAD,), jnp.float32)
    src2d = src.reshape(E_PAD // K, K)
    dst2d = dst.reshape(E_PAD // K, K)
    num_part, den_part = sc_layer2(src2d, dst2d, aux[:, 0], aux[:, 1], hw2,
                                   zrows, zden)

    den = den_part[0, :N] + den_part[1, :N] + 1e-16
    den_b = jnp.broadcast_to(den[:, None], (N, CH))

    wo_pad = jnp.concatenate(
        [W_out, jnp.zeros((CH, CH - OUT_CH), jnp.float32)], axis=1)
    bo_pad = jnp.concatenate(
        [b_out, jnp.zeros((CH - OUT_CH,), jnp.float32)]).reshape(1, CH)

    outp = _tc_final(num_part, den_b, b2.reshape(1, CH),
                     batch.astype(jnp.int32).reshape(NB, 1, R),
                     wo_pad, bo_pad)
    return outp[:, :OUT_CH]


# speedup vs baseline: 1.7530x; 1.7530x over previous
import functools

import jax
import jax.numpy as jnp
from jax import lax
from jax.experimental import pallas as pl
from jax.experimental.pallas import tpu as pltpu
from jax.experimental.pallas import tpu_sc as plsc

N = 10000
E = 320000
CH = 128
OUT_CH = 64
NUM_GRAPHS = 64

NC = 2
NS = 16
NW = NC * NS
K = 96

E_TOT = E + N
G = 8
S = G * K
NCH2 = -(-E_TOT // (NW * K))
NCH2 = NCH2 + (NCH2 % 2)
NCH1 = -(-NCH2 // (2 * G)) * 2 * G
NSUP = NCH1 // G
E_PAD = NW * K * NCH1
EPW1 = K * NCH1
EPW2 = K * NCH2

N_PAD = 10240
ROWS_PT = N_PAD // NS
DEN_PT = N_PAD // NS

def _sc_layer1_body(src_hbm, dst_hbm, x_hbm, tst_hbm, zflat_hbm, out_hbm,
                    x_v, ts_v, td_v,
                    sbigA, dbigA, wgA, idxgA, semLA, semSA,
                    sbigB, dbigB, wgB, idxgB, semLB, semSB,
                    c_acc):
    cid = lax.axis_index("c")
    sid = lax.axis_index("s")
    wid = cid * NS + sid

    pltpu.sync_copy(x_hbm, x_v)
    pltpu.sync_copy(tst_hbm.at[0], ts_v)
    pltpu.sync_copy(tst_hbm.at[1], td_v)
    seg = (N * CH) // NS
    pltpu.sync_copy(zflat_hbm.at[pl.ds(sid * seg, seg)],
                    c_acc.at[pl.ds(sid * seg, seg)])
    plsc.subcore_barrier()

    def issue_loads(u, sb, db, semL):
        base = wid * EPW1 + u * S
        pltpu.async_copy(src_hbm.at[pl.ds(base, S)], sb, semL)
        pltpu.async_copy(dst_hbm.at[pl.ds(base, S)], db, semL)

    def wait_loads(u, sb, db, semL):
        base = wid * EPW1 + u * S
        pltpu.make_async_copy(src_hbm.at[pl.ds(base, S)], sb, semL).wait()
        pltpu.make_async_copy(dst_hbm.at[pl.ds(base, S)], db, semL).wait()

    def process_super(u, sb, db, wg, idxg, semS):
        base = wid * EPW1 + u * S
        for g in range(G):
            for j in range(K // 16):
                off = g * K + j * 16
                sl = sb[pl.ds(off, 16)]
                d = db[pl.ds(off, 16)]
                cs = plsc.load_gather(x_v, [sl])
                cd = plsc.load_gather(x_v, [d])
                a = plsc.load_gather(ts_v, [cs])
                b = plsc.load_gather(td_v, [cd])
                e = a + b
                e = jnp.where(e > 0.0, e, 0.2 * e)
                w = jnp.exp(e)
                eid = base + off + lax.iota(jnp.int32, 16)
                w = jnp.where(eid < E_TOT, w, 0.0)
                wg[g, pl.ds(j * 16, 16)] = w
                idxg[g, pl.ds(j * 16, 16)] = d * CH + cs
            pltpu.async_copy(wg.at[g], c_acc.at[idxg.at[g]], semS, add=True)

    def drain_scatters(wg, idxg, semS):
        for g in range(G):
            pltpu.make_async_copy(wg.at[g], c_acc.at[idxg.at[g]], semS).wait()

    issue_loads(0, sbigA, dbigA, semLA)

    def pair(u2, carry):
        uA = u2 * 2
        uB = uA + 1
        uA2 = lax.rem(uA + 2, NSUP)
        issue_loads(uB, sbigB, dbigB, semLB)
        wait_loads(uA, sbigA, dbigA, semLA)
        process_super(uA, sbigA, dbigA, wgA, idxgA, semSA)
        issue_loads(uA2, sbigA, dbigA, semLA)
        drain_scatters(wgA, idxgA, semSA)
        wait_loads(uB, sbigB, dbigB, semLB)
        process_super(uB, sbigB, dbigB, wgB, idxgB, semSB)
        drain_scatters(wgB, idxgB, semSB)
        return carry

    lax.fori_loop(0, NSUP // 2, pair, 0)
    wait_loads(0, sbigA, dbigA, semLA)
    plsc.subcore_barrier()
    pltpu.sync_copy(c_acc.at[pl.ds(sid * seg, seg)],
                    out_hbm.at[cid, pl.ds(sid * seg, seg)])


def _sc_layer2_body(ed2d_hbm, as_hbm, ad_hbm, hw2_hbm, zrows_hbm,
                    zden_hbm, num_out, den_out,
                    as_v, ad_v,
                    sdA, dstA, wA, rowsA, semA,
                    sdB, dstB, wB, rowsB, semB,
                    num_acc, den_acc):
    cid = lax.axis_index("c")
    sid = lax.axis_index("s")
    wid = cid * NS + sid

    pltpu.sync_copy(as_hbm, as_v)
    pltpu.sync_copy(ad_hbm, ad_v)
    pltpu.sync_copy(zrows_hbm.at[pl.ds(sid * ROWS_PT, ROWS_PT)],
                    num_acc.at[pl.ds(sid * ROWS_PT, ROWS_PT)])
    pltpu.sync_copy(zden_hbm.at[pl.ds(sid * DEN_PT, DEN_PT)],
                    den_acc.at[pl.ds(sid * DEN_PT, DEN_PT)])
    plsc.subcore_barrier()

    def load_idx(c, sdv, dv):
        pltpu.sync_copy(ed2d_hbm.at[wid * NCH2 + c], sdv)
        for j in range(K // 16):
            dv[pl.ds(j * 16, 16)] = sdv[pl.ds(K + j * 16, 16)]

    def process(c, sdv, dv, wv, rv):
        base = wid * EPW2 + c * K
        for j in range(K // 16):
            s = sdv[pl.ds(j * 16, 16)]
            d = dv[pl.ds(j * 16, 16)]
            a = plsc.load_gather(as_v, [s])
            b = plsc.load_gather(ad_v, [d])
            e = a + b
            e = jnp.where(e > 0.0, e, 0.2 * e)
            w = jnp.exp(e)
            eid = base + j * 16 + lax.iota(jnp.int32, 16)
            w = jnp.where(eid < E_TOT, w, 0.0)
            wv[pl.ds(j * 16, 16)] = w

        def scale_row(r, carry2):
            wr = plsc.load_gather(wv, [jnp.full((16,), r, jnp.int32)])
            for l in range(CH // 16):
                rv[r, pl.ds(l * 16, 16)] = rv[r, pl.ds(l * 16, 16)] * wr
            return carry2

        lax.fori_loop(0, K, scale_row, 0, unroll=8)
        pltpu.sync_copy(rv, num_acc.at[dv], add=True)
        pltpu.sync_copy(wv, den_acc.at[dv], add=True)

    load_idx(0, sdA, dstA)
    pltpu.async_copy(hw2_hbm.at[sdA.at[pl.ds(0, K)]], rowsA, semA)

    def pair(i, carry):
        c0 = i * 2
        c1 = c0 + 1
        c2 = lax.rem(c0 + 2, NCH2)
        load_idx(c1, sdB, dstB)
        pltpu.async_copy(hw2_hbm.at[sdB.at[pl.ds(0, K)]], rowsB, semB)
        pltpu.make_async_copy(hw2_hbm.at[sdA.at[pl.ds(0, K)]], rowsA,
                              semA).wait()
        process(c0, sdA, dstA, wA, rowsA)
        load_idx(c2, sdA, dstA)
        pltpu.async_copy(hw2_hbm.at[sdA.at[pl.ds(0, K)]], rowsA, semA)
        pltpu.make_async_copy(hw2_hbm.at[sdB.at[pl.ds(0, K)]], rowsB,
                              semB).wait()
        process(c1, sdB, dstB, wB, rowsB)
        return carry

    lax.fori_loop(0, NCH2 // 2, pair, 0)
    pltpu.make_async_copy(hw2_hbm.at[sdA.at[pl.ds(0, K)]], rowsA, semA).wait()
    plsc.subcore_barrier()
    pltpu.sync_copy(num_acc.at[pl.ds(sid * ROWS_PT, ROWS_PT)],
                    num_out.at[cid, pl.ds(sid * ROWS_PT, ROWS_PT)])
    pltpu.sync_copy(den_acc.at[pl.ds(sid * DEN_PT, DEN_PT)],
                    den_out.at[cid, pl.ds(sid * DEN_PT, DEN_PT)])


@functools.lru_cache(maxsize=None)
def _sc_kernels():
    mesh = plsc.VectorSubcoreMesh(core_axis_name="c", subcore_axis_name="s",
                                  num_cores=NC, num_subcores=NS)
    cparams = pltpu.CompilerParams(needs_layout_passes=False)
    layer1 = pl.kernel(
        _sc_layer1_body,
        out_type=jax.ShapeDtypeStruct((NC, N * CH), jnp.float32),
        mesh=mesh,
        compiler_params=cparams,
        scratch_types=[
            pltpu.VMEM((N,), jnp.int32),
            pltpu.VMEM((CH,), jnp.float32),
            pltpu.VMEM((CH,), jnp.float32),
            pltpu.VMEM((S,), jnp.int32),
            pltpu.VMEM((S,), jnp.int32),
            pltpu.VMEM((G, K), jnp.float32),
            pltpu.VMEM((G, K), jnp.int32),
            pltpu.SemaphoreType.DMA,
            pltpu.SemaphoreType.DMA,
            pltpu.VMEM((S,), jnp.int32),
            pltpu.VMEM((S,), jnp.int32),
            pltpu.VMEM((G, K), jnp.float32),
            pltpu.VMEM((G, K), jnp.int32),
            pltpu.SemaphoreType.DMA,
            pltpu.SemaphoreType.DMA,
            pltpu.VMEM_SHARED((N * CH,), jnp.float32),
        ],
    )
    layer2 = pl.kernel(
        _sc_layer2_body,
        out_type=[jax.ShapeDtypeStruct((NC, N_PAD, CH), jnp.float32),
                  jax.ShapeDtypeStruct((NC, N_PAD), jnp.float32)],
        mesh=mesh,
        compiler_params=cparams,
        scratch_types=[
            pltpu.VMEM((N,), jnp.float32),
            pltpu.VMEM((N,), jnp.float32),
            pltpu.VMEM((2 * K,), jnp.int32),
            pltpu.VMEM((K,), jnp.int32),
            pltpu.VMEM((K,), jnp.float32),
            pltpu.VMEM((K, CH), jnp.float32),
            pltpu.SemaphoreType.DMA,
            pltpu.VMEM((2 * K,), jnp.int32),
            pltpu.VMEM((K,), jnp.int32),
            pltpu.VMEM((K,), jnp.float32),
            pltpu.VMEM((K, CH), jnp.float32),
            pltpu.SemaphoreType.DMA,
            pltpu.VMEM_SHARED((N_PAD, CH), jnp.float32),
            pltpu.VMEM_SHARED((N_PAD,), jnp.float32),
        ],
    )
    return layer1, layer2


def _prep_body(emb_ref, w1_ref, as1_ref, ad1_ref, tst_ref, m1_ref):
    m1 = jnp.dot(emb_ref[...], w1_ref[...], preferred_element_type=jnp.float32)
    m1_ref[...] = m1
    dn = (((1,), (1,)), ((), ()))
    tst_ref[0:1, :] = lax.dot_general(as1_ref[...], m1, dn,
                                      preferred_element_type=jnp.float32)
    tst_ref[1:2, :] = lax.dot_general(ad1_ref[...], m1, dn,
                                      preferred_element_type=jnp.float32)


def _tc_prep(emb, w1, as1, ad1):
    return pl.pallas_call(
        _prep_body,
        out_shape=[jax.ShapeDtypeStruct((2, CH), jnp.float32),
                   jax.ShapeDtypeStruct((CH, CH), jnp.float32)],
    )(emb, w1, as1, ad1)


R = 1000
NB = N // R


def _mid_body(c_ref, m1_ref, b1_ref, w2_ref, as2_ref, ad2_ref,
              hw2_ref, aux_ref):
    c = c_ref[0] + c_ref[1]
    den = jnp.sum(c, axis=1, keepdims=True) + 1e-16
    h1 = jnp.dot(c, m1_ref[...], preferred_element_type=jnp.float32) / den
    h1 = h1 + b1_ref[...]
    h2 = jnp.maximum(h1, 0.0)
    hw2 = jnp.dot(h2, w2_ref[...], preferred_element_type=jnp.float32)
    hw2_ref[...] = hw2
    a_s = jnp.sum(hw2 * as2_ref[...], axis=1, keepdims=True)
    a_d = jnp.sum(hw2 * ad2_ref[...], axis=1, keepdims=True)
    lane = lax.broadcasted_iota(jnp.int32, (1, CH), 1)
    aux_ref[...] = (jnp.where(lane == 0, a_s, 0.0)
                    + jnp.where(lane == 1, a_d, 0.0))


def _tc_mid(c3, m1, b1, w2, as2, ad2):
    return pl.pallas_call(
        _mid_body,
        grid=(NB,),
        in_specs=[
            pl.BlockSpec((NC, R, CH), lambda i: (0, i, 0)),
            pl.BlockSpec((CH, CH), lambda i: (0, 0)),
            pl.BlockSpec((1, CH), lambda i: (0, 0)),
            pl.BlockSpec((CH, CH), lambda i: (0, 0)),
            pl.BlockSpec((1, CH), lambda i: (0, 0)),
            pl.BlockSpec((1, CH), lambda i: (0, 0)),
        ],
        out_specs=[
            pl.BlockSpec((R, CH), lambda i: (i, 0)),
            pl.BlockSpec((R, CH), lambda i: (i, 0)),
        ],
        out_shape=[jax.ShapeDtypeStruct((N, CH), jnp.float32),
                   jax.ShapeDtypeStruct((N, CH), jnp.float32)],
    )(c3, m1, b1, w2, as2, ad2)


def _fin_body(num_ref, den_ref, b2_ref, batch_ref, wo_ref, bo_ref, out_ref,
              acc_sum, acc_cnt):
    i = pl.program_id(0)

    @pl.when(i == 0)
    def _():
        acc_sum[...] = jnp.zeros_like(acc_sum)
        acc_cnt[...] = jnp.zeros_like(acc_cnt)

    h = (num_ref[0] + num_ref[1]) / den_ref[...] + b2_ref[...]
    bvec = batch_ref[0]
    gids = lax.broadcasted_iota(jnp.int32, (NUM_GRAPHS, R), 0)
    p = (gids == bvec).astype(jnp.float32)
    acc_sum[...] += jnp.dot(p, h, preferred_element_type=jnp.float32)
    acc_cnt[...] += jnp.sum(p, axis=1, keepdims=True)

    @pl.when(i == NB - 1)
    def _():
        pooled = acc_sum[...] / jnp.maximum(acc_cnt[...], 1.0)
        out_ref[...] = (jnp.dot(pooled, wo_ref[...],
                                preferred_element_type=jnp.float32)
                        + bo_ref[...])


def _tc_final(num3, den_b, b2, batch3, wo_pad, bo_pad):
    return pl.pallas_call(
        _fin_body,
        grid=(NB,),
        in_specs=[
            pl.BlockSpec((NC, R, CH), lambda i: (0, i, 0)),
            pl.BlockSpec((R, CH), lambda i: (i, 0)),
            pl.BlockSpec((1, CH), lambda i: (0, 0)),
            pl.BlockSpec((1, 1, R), lambda i: (i, 0, 0)),
            pl.BlockSpec((CH, CH), lambda i: (0, 0)),
            pl.BlockSpec((1, CH), lambda i: (0, 0)),
        ],
        out_specs=pl.BlockSpec((NUM_GRAPHS, CH), lambda i: (0, 0)),
        out_shape=jax.ShapeDtypeStruct((NUM_GRAPHS, CH), jnp.float32),
        scratch_shapes=[pltpu.VMEM((NUM_GRAPHS, CH), jnp.float32),
                        pltpu.VMEM((NUM_GRAPHS, CH), jnp.float32)],
    )(num3, den_b, b2, batch3, wo_pad, bo_pad)


def kernel(x, edge_index, batch, emb_table, W1, a_src1, a_dst1, b1,
           W2, a_src2, a_dst2, b2, W_out, b_out):
    x = x.astype(jnp.int32)
    loop = jnp.arange(N, dtype=jnp.int32)
    padz = jnp.zeros((E_PAD - E_TOT,), jnp.int32)
    src = jnp.concatenate([edge_index[0].astype(jnp.int32), loop, padz])
    dst = jnp.concatenate([edge_index[1].astype(jnp.int32), loop, padz])

    sc_layer1, sc_layer2 = _sc_kernels()

    tst, m1 = _tc_prep(emb_table, W1,
                       a_src1.reshape(1, CH), a_dst1.reshape(1, CH))

    zflat = jnp.zeros((N * CH,), jnp.float32)
    c_part = sc_layer1(src, dst, x, tst, zflat)

    hw2, aux = _tc_mid(c_part.reshape(NC, N, CH), m1, b1.reshape(1, CH),
                       W2, a_src2.reshape(1, CH), a_dst2.reshape(1, CH))

    zrows = jnp.zeros((N_PAD, CH), jnp.float32)
    zden = jnp.zeros((N_PAD,), jnp.float32)
    ed2d = jnp.concatenate([src.reshape(E_PAD // K, K)[:NW * NCH2],
                            dst.reshape(E_PAD // K, K)[:NW * NCH2]], axis=1)
    num_part, den_part = sc_layer2(ed2d, aux[:, 0], aux[:, 1], hw2,
                                   zrows, zden)
    num_part = num_part[:, :N, :]

    den = den_part[0, :N] + den_part[1, :N] + 1e-16
    den_b = jnp.broadcast_to(den[:, None], (N, CH))

    wo_pad = jnp.concatenate(
        [W_out, jnp.zeros((CH, CH - OUT_CH), jnp.float32)], axis=1)
    bo_pad = jnp.concatenate(
        [b_out, jnp.zeros((CH - OUT_CH,), jnp.float32)]).reshape(1, CH)

    outp = _tc_final(num_part, den_b, b2.reshape(1, CH),
                     batch.astype(jnp.int32).reshape(NB, 1, R),
                     wo_pad, bo_pad)
    return outp[:, :OUT_CH]


# speedup vs baseline: 1.9384x; 1.1058x over previous
import functools

import jax
import jax.numpy as jnp
from jax import lax
from jax.experimental import pallas as pl
from jax.experimental.pallas import tpu as pltpu
from jax.experimental.pallas import tpu_sc as plsc

N = 10000
E = 320000
CH = 128
OUT_CH = 64
NUM_GRAPHS = 64

NC = 2
NS = 16
NW = NC * NS
K = 96

E_TOT = E + N
G = 8
S = G * K
NCH2 = -(-E_TOT // (NW * K))
NCH2 = NCH2 + (NCH2 % 2)
NCH1 = -(-NCH2 // (2 * G)) * 2 * G
NSUP = NCH1 // G
E_PAD = NW * K * NCH1
EPW1 = K * NCH1
EPW2 = K * NCH2

N_PAD = 10240
ROWS_PT = N_PAD // NS
DEN_PT = N_PAD // NS

def _sc_layer1_body(src_hbm, dst_hbm, x_hbm, tst_hbm, zflat_hbm, out_hbm,
                    x_v, ts_v, td_v,
                    sbigA, dbigA, wgA, idxgA, semLA, semSA,
                    sbigB, dbigB, wgB, idxgB, semLB, semSB,
                    c_acc):
    cid = lax.axis_index("c")
    sid = lax.axis_index("s")
    wid = cid * NS + sid

    pltpu.sync_copy(x_hbm, x_v)
    pltpu.sync_copy(tst_hbm.at[0], ts_v)
    pltpu.sync_copy(tst_hbm.at[1], td_v)
    seg = (N * CH) // NS
    pltpu.sync_copy(zflat_hbm.at[pl.ds(sid * seg, seg)],
                    c_acc.at[pl.ds(sid * seg, seg)])
    plsc.subcore_barrier()

    def issue_loads(u, sb, db, semL):
        base = wid * EPW1 + u * S
        pltpu.async_copy(src_hbm.at[pl.ds(base, S)], sb, semL)
        pltpu.async_copy(dst_hbm.at[pl.ds(base, S)], db, semL)

    def wait_loads(u, sb, db, semL):
        base = wid * EPW1 + u * S
        pltpu.make_async_copy(src_hbm.at[pl.ds(base, S)], sb, semL).wait()
        pltpu.make_async_copy(dst_hbm.at[pl.ds(base, S)], db, semL).wait()

    def process_super(u, sb, db, wg, idxg, semS):
        base = wid * EPW1 + u * S
        for g in range(G):
            for j in range(K // 16):
                off = g * K + j * 16
                sl = sb[pl.ds(off, 16)]
                d = db[pl.ds(off, 16)]
                cs = plsc.load_gather(x_v, [sl])
                cd = plsc.load_gather(x_v, [d])
                a = plsc.load_gather(ts_v, [cs])
                b = plsc.load_gather(td_v, [cd])
                e = a + b
                e = jnp.where(e > 0.0, e, 0.2 * e)
                w = jnp.exp(e)
                eid = base + off + lax.iota(jnp.int32, 16)
                w = jnp.where(eid < E_TOT, w, 0.0)
                wg[g, pl.ds(j * 16, 16)] = w
                idxg[g, pl.ds(j * 16, 16)] = d * CH + cs
            pltpu.async_copy(wg.at[g], c_acc.at[idxg.at[g]], semS, add=True)

    def drain_scatters(wg, idxg, semS):
        for g in range(G):
            pltpu.make_async_copy(wg.at[g], c_acc.at[idxg.at[g]], semS).wait()

    issue_loads(0, sbigA, dbigA, semLA)

    def pair(u2, carry):
        uA = u2 * 2
        uB = uA + 1
        uA2 = lax.rem(uA + 2, NSUP)
        issue_loads(uB, sbigB, dbigB, semLB)
        wait_loads(uA, sbigA, dbigA, semLA)
        process_super(uA, sbigA, dbigA, wgA, idxgA, semSA)
        issue_loads(uA2, sbigA, dbigA, semLA)
        drain_scatters(wgA, idxgA, semSA)
        wait_loads(uB, sbigB, dbigB, semLB)
        process_super(uB, sbigB, dbigB, wgB, idxgB, semSB)
        drain_scatters(wgB, idxgB, semSB)
        return carry

    lax.fori_loop(0, NSUP // 2, pair, 0)
    wait_loads(0, sbigA, dbigA, semLA)
    plsc.subcore_barrier()
    pltpu.sync_copy(c_acc.at[pl.ds(sid * seg, seg)],
                    out_hbm.at[cid, pl.ds(sid * seg, seg)])


def _sc_layer2_body(ed2d_hbm, as_hbm, ad_hbm, hw2_hbm, zrows_hbm,
                    zden_hbm, num_out, den_out,
                    as_v, ad_v,
                    sdA, dstA, wA, rowsA, semA, semSA,
                    sdB, dstB, wB, rowsB, semB, semSB,
                    num_acc, den_acc):
    cid = lax.axis_index("c")
    sid = lax.axis_index("s")
    wid = cid * NS + sid

    pltpu.sync_copy(as_hbm, as_v)
    pltpu.sync_copy(ad_hbm, ad_v)
    pltpu.sync_copy(zrows_hbm.at[pl.ds(sid * ROWS_PT, ROWS_PT)],
                    num_acc.at[pl.ds(sid * ROWS_PT, ROWS_PT)])
    pltpu.sync_copy(zden_hbm.at[pl.ds(sid * DEN_PT, DEN_PT)],
                    den_acc.at[pl.ds(sid * DEN_PT, DEN_PT)])
    plsc.subcore_barrier()

    def load_sd(c, sdv):
        pltpu.sync_copy(ed2d_hbm.at[wid * NCH2 + c], sdv)

    def stage_dst(sdv, dv):
        for j in range(K // 16):
            dv[pl.ds(j * 16, 16)] = sdv[pl.ds(K + j * 16, 16)]

    def issue_scatters(dv, wv, rv, semS):
        pltpu.async_copy(rv, num_acc.at[dv], semS, add=True)
        pltpu.async_copy(wv, den_acc.at[dv], semS, add=True)

    def drain_scatters(dv, wv, rv, semS):
        pltpu.make_async_copy(rv, num_acc.at[dv], semS).wait()
        pltpu.make_async_copy(wv, den_acc.at[dv], semS).wait()

    def process(c, sdv, dv, wv, rv):
        base = wid * EPW2 + c * K
        for j in range(K // 16):
            s = sdv[pl.ds(j * 16, 16)]
            d = dv[pl.ds(j * 16, 16)]
            a = plsc.load_gather(as_v, [s])
            b = plsc.load_gather(ad_v, [d])
            e = a + b
            e = jnp.where(e > 0.0, e, 0.2 * e)
            w = jnp.exp(e)
            eid = base + j * 16 + lax.iota(jnp.int32, 16)
            w = jnp.where(eid < E_TOT, w, 0.0)
            wv[pl.ds(j * 16, 16)] = w

        def scale_row(r, carry2):
            wr = plsc.load_gather(wv, [jnp.full((16,), r, jnp.int32)])
            for l in range(CH // 16):
                rv[r, pl.ds(l * 16, 16)] = rv[r, pl.ds(l * 16, 16)] * wr
            return carry2

        lax.fori_loop(0, K, scale_row, 0, unroll=8)

    load_sd(0, sdA)
    stage_dst(sdA, dstA)
    stage_dst(sdA, dstB)
    pltpu.sync_copy(zden_hbm.at[pl.ds(0, K)], wB)
    pltpu.sync_copy(zrows_hbm.at[pl.ds(0, K)], rowsB)
    issue_scatters(dstB, wB, rowsB, semSB)
    pltpu.async_copy(hw2_hbm.at[sdA.at[pl.ds(0, K)]], rowsA, semA)

    def pair(i, carry):
        c0 = i * 2
        c1 = c0 + 1
        c2 = lax.rem(c0 + 2, NCH2)
        load_sd(c1, sdB)
        drain_scatters(dstB, wB, rowsB, semSB)
        stage_dst(sdB, dstB)
        pltpu.async_copy(hw2_hbm.at[sdB.at[pl.ds(0, K)]], rowsB, semB)
        pltpu.make_async_copy(hw2_hbm.at[sdA.at[pl.ds(0, K)]], rowsA,
                              semA).wait()
        process(c0, sdA, dstA, wA, rowsA)
        issue_scatters(dstA, wA, rowsA, semSA)
        load_sd(c2, sdA)
        drain_scatters(dstA, wA, rowsA, semSA)
        stage_dst(sdA, dstA)
        pltpu.async_copy(hw2_hbm.at[sdA.at[pl.ds(0, K)]], rowsA, semA)
        pltpu.make_async_copy(hw2_hbm.at[sdB.at[pl.ds(0, K)]], rowsB,
                              semB).wait()
        process(c1, sdB, dstB, wB, rowsB)
        issue_scatters(dstB, wB, rowsB, semSB)
        return carry

    lax.fori_loop(0, NCH2 // 2, pair, 0)
    pltpu.make_async_copy(hw2_hbm.at[sdA.at[pl.ds(0, K)]], rowsA, semA).wait()
    drain_scatters(dstB, wB, rowsB, semSB)
    plsc.subcore_barrier()
    pltpu.sync_copy(num_acc.at[pl.ds(sid * ROWS_PT, ROWS_PT)],
                    num_out.at[cid, pl.ds(sid * ROWS_PT, ROWS_PT)])
    pltpu.sync_copy(den_acc.at[pl.ds(sid * DEN_PT, DEN_PT)],
                    den_out.at[cid, pl.ds(sid * DEN_PT, DEN_PT)])


@functools.lru_cache(maxsize=None)
def _sc_kernels():
    mesh = plsc.VectorSubcoreMesh(core_axis_name="c", subcore_axis_name="s",
                                  num_cores=NC, num_subcores=NS)
    cparams = pltpu.CompilerParams(needs_layout_passes=False)
    layer1 = pl.kernel(
        _sc_layer1_body,
        out_type=jax.ShapeDtypeStruct((NC, N * CH), jnp.float32),
        mesh=mesh,
        compiler_params=cparams,
        scratch_types=[
            pltpu.VMEM((N,), jnp.int32),
            pltpu.VMEM((CH,), jnp.float32),
            pltpu.VMEM((CH,), jnp.float32),
            pltpu.VMEM((S,), jnp.int32),
            pltpu.VMEM((S,), jnp.int32),
            pltpu.VMEM((G, K), jnp.float32),
            pltpu.VMEM((G, K), jnp.int32),
            pltpu.SemaphoreType.DMA,
            pltpu.SemaphoreType.DMA,
            pltpu.VMEM((S,), jnp.int32),
            pltpu.VMEM((S,), jnp.int32),
            pltpu.VMEM((G, K), jnp.float32),
            pltpu.VMEM((G, K), jnp.int32),
            pltpu.SemaphoreType.DMA,
            pltpu.SemaphoreType.DMA,
            pltpu.VMEM_SHARED((N * CH,), jnp.float32),
        ],
    )
    layer2 = pl.kernel(
        _sc_layer2_body,
        out_type=[jax.ShapeDtypeStruct((NC, N_PAD, CH), jnp.float32),
                  jax.ShapeDtypeStruct((NC, N_PAD), jnp.float32)],
        mesh=mesh,
        compiler_params=cparams,
        scratch_types=[
            pltpu.VMEM((N,), jnp.float32),
            pltpu.VMEM((N,), jnp.float32),
            pltpu.VMEM((2 * K,), jnp.int32),
            pltpu.VMEM((K,), jnp.int32),
            pltpu.VMEM((K,), jnp.float32),
            pltpu.VMEM((K, CH), jnp.float32),
            pltpu.SemaphoreType.DMA,
            pltpu.SemaphoreType.DMA,
            pltpu.VMEM((2 * K,), jnp.int32),
            pltpu.VMEM((K,), jnp.int32),
            pltpu.VMEM((K,), jnp.float32),
            pltpu.VMEM((K, CH), jnp.float32),
            pltpu.SemaphoreType.DMA,
            pltpu.SemaphoreType.DMA,
            pltpu.VMEM_SHARED((N_PAD, CH), jnp.float32),
            pltpu.VMEM_SHARED((N_PAD,), jnp.float32),
        ],
    )
    return layer1, layer2


def _prep_body(emb_ref, w1_ref, as1_ref, ad1_ref, tst_ref, m1_ref):
    m1 = jnp.dot(emb_ref[...], w1_ref[...], preferred_element_type=jnp.float32)
    m1_ref[...] = m1
    dn = (((1,), (1,)), ((), ()))
    tst_ref[0:1, :] = lax.dot_general(as1_ref[...], m1, dn,
                                      preferred_element_type=jnp.float32)
    tst_ref[1:2, :] = lax.dot_general(ad1_ref[...], m1, dn,
                                      preferred_element_type=jnp.float32)


def _tc_prep(emb, w1, as1, ad1):
    return pl.pallas_call(
        _prep_body,
        out_shape=[jax.ShapeDtypeStruct((2, CH), jnp.float32),
                   jax.ShapeDtypeStruct((CH, CH), jnp.float32)],
    )(emb, w1, as1, ad1)


R = 1000
NB = N // R


def _mid_body(c_ref, m1_ref, b1_ref, w2_ref, as2_ref, ad2_ref,
              hw2_ref, aux_ref):
    c = c_ref[0] + c_ref[1]
    den = jnp.sum(c, axis=1, keepdims=True) + 1e-16
    h1 = jnp.dot(c, m1_ref[...], preferred_element_type=jnp.float32) / den
    h1 = h1 + b1_ref[...]
    h2 = jnp.maximum(h1, 0.0)
    hw2 = jnp.dot(h2, w2_ref[...], preferred_element_type=jnp.float32)
    hw2_ref[...] = hw2
    a_s = jnp.sum(hw2 * as2_ref[...], axis=1, keepdims=True)
    a_d = jnp.sum(hw2 * ad2_ref[...], axis=1, keepdims=True)
    lane = lax.broadcasted_iota(jnp.int32, (1, CH), 1)
    aux_ref[...] = (jnp.where(lane == 0, a_s, 0.0)
                    + jnp.where(lane == 1, a_d, 0.0))


def _tc_mid(c3, m1, b1, w2, as2, ad2):
    return pl.pallas_call(
        _mid_body,
        grid=(NB,),
        in_specs=[
            pl.BlockSpec((NC, R, CH), lambda i: (0, i, 0)),
            pl.BlockSpec((CH, CH), lambda i: (0, 0)),
            pl.BlockSpec((1, CH), lambda i: (0, 0)),
            pl.BlockSpec((CH, CH), lambda i: (0, 0)),
            pl.BlockSpec((1, CH), lambda i: (0, 0)),
            pl.BlockSpec((1, CH), lambda i: (0, 0)),
        ],
        out_specs=[
            pl.BlockSpec((R, CH), lambda i: (i, 0)),
            pl.BlockSpec((R, CH), lambda i: (i, 0)),
        ],
        out_shape=[jax.ShapeDtypeStruct((N, CH), jnp.float32),
                   jax.ShapeDtypeStruct((N, CH), jnp.float32)],
    )(c3, m1, b1, w2, as2, ad2)


def _fin_body(num_ref, den_ref, b2_ref, batch_ref, wo_ref, bo_ref, out_ref,
              acc_sum, acc_cnt):
    i = pl.program_id(0)

    @pl.when(i == 0)
    def _():
        acc_sum[...] = jnp.zeros_like(acc_sum)
        acc_cnt[...] = jnp.zeros_like(acc_cnt)

    h = (num_ref[0] + num_ref[1]) / den_ref[...] + b2_ref[...]
    bvec = batch_ref[0]
    gids = lax.broadcasted_iota(jnp.int32, (NUM_GRAPHS, R), 0)
    p = (gids == bvec).astype(jnp.float32)
    acc_sum[...] += jnp.dot(p, h, preferred_element_type=jnp.float32)
    acc_cnt[...] += jnp.sum(p, axis=1, keepdims=True)

    @pl.when(i == NB - 1)
    def _():
        pooled = acc_sum[...] / jnp.maximum(acc_cnt[...], 1.0)
        out_ref[...] = (jnp.dot(pooled, wo_ref[...],
                                preferred_element_type=jnp.float32)
                        + bo_ref[...])


def _tc_final(num3, den_b, b2, batch3, wo_pad, bo_pad):
    return pl.pallas_call(
        _fin_body,
        grid=(NB,),
        in_specs=[
            pl.BlockSpec((NC, R, CH), lambda i: (0, i, 0)),
            pl.BlockSpec((R, CH), lambda i: (i, 0)),
            pl.BlockSpec((1, CH), lambda i: (0, 0)),
            pl.BlockSpec((1, 1, R), lambda i: (i, 0, 0)),
            pl.BlockSpec((CH, CH), lambda i: (0, 0)),
            pl.BlockSpec((1, CH), lambda i: (0, 0)),
        ],
        out_specs=pl.BlockSpec((NUM_GRAPHS, CH), lambda i: (0, 0)),
        out_shape=jax.ShapeDtypeStruct((NUM_GRAPHS, CH), jnp.float32),
        scratch_shapes=[pltpu.VMEM((NUM_GRAPHS, CH), jnp.float32),
                        pltpu.VMEM((NUM_GRAPHS, CH), jnp.float32)],
    )(num3, den_b, b2, batch3, wo_pad, bo_pad)


def kernel(x, edge_index, batch, emb_table, W1, a_src1, a_dst1, b1,
           W2, a_src2, a_dst2, b2, W_out, b_out):
    x = x.astype(jnp.int32)
    loop = jnp.arange(N, dtype=jnp.int32)
    padz = jnp.zeros((E_PAD - E_TOT,), jnp.int32)
    src = jnp.concatenate([edge_index[0].astype(jnp.int32), loop, padz])
    dst = jnp.concatenate([edge_index[1].astype(jnp.int32), loop, padz])

    sc_layer1, sc_layer2 = _sc_kernels()

    tst, m1 = _tc_prep(emb_table, W1,
                       a_src1.reshape(1, CH), a_dst1.reshape(1, CH))

    zflat = jnp.zeros((N * CH,), jnp.float32)
    c_part = sc_layer1(src, dst, x, tst, zflat)

    hw2, aux = _tc_mid(c_part.reshape(NC, N, CH), m1, b1.reshape(1, CH),
                       W2, a_src2.reshape(1, CH), a_dst2.reshape(1, CH))

    zrows = jnp.zeros((N_PAD, CH), jnp.float32)
    zden = jnp.zeros((N_PAD,), jnp.float32)
    ed2d = jnp.concatenate([src.reshape(E_PAD // K, K)[:NW * NCH2],
                            dst.reshape(E_PAD // K, K)[:NW * NCH2]], axis=1)
    num_part, den_part = sc_layer2(ed2d, aux[:, 0], aux[:, 1], hw2,
                                   zrows, zden)
    num_part = num_part[:, :N, :]

    den = den_part[0, :N] + den_part[1, :N] + 1e-16
    den_b = jnp.broadcast_to(den[:, None], (N, CH))

    wo_pad = jnp.concatenate(
        [W_out, jnp.zeros((CH, CH - OUT_CH), jnp.float32)], axis=1)
    bo_pad = jnp.concatenate(
        [b_out, jnp.zeros((CH - OUT_CH,), jnp.float32)]).reshape(1, CH)

    outp = _tc_final(num_part, den_b, b2.reshape(1, CH),
                     batch.astype(jnp.int32).reshape(NB, 1, R),
                     wo_pad, bo_pad)
    return outp[:, :OUT_CH]
